# Initial kernel scaffold; baseline (speedup 1.0000x reference)
import jax
import jax.numpy as jnp
from jax.experimental import pallas as pl


def kernel(query, key, value):
    raise NotImplementedError("write your pallas kernel here")


# speedup vs baseline: 1.7240x; 1.7240x over previous
import jax
import jax.numpy as jnp
from jax import lax
from jax.experimental import pallas as pl
from jax.experimental.pallas import tpu as pltpu
from jax.experimental.pallas import tpu_sc as plsc

B, C, N = 8, 768, 1024
GROUP_SIZES = (96, 96, 192, 384)
FLATOFF = (0, 768, 1536, 3072)
NW = 32
CNT = tuple(8 * gs // NW for gs in GROUP_SIZES)


def _xm_body(q_ref, xm_ref):
    q = q_ref[...]
    cf = jnp.mean(q, axis=0)
    rm = jnp.mean(cf, axis=1, keepdims=True)
    xm_ref[...] = cf - rm


_CB = 128
_xm_call = pl.pallas_call(
    _xm_body,
    grid=(C // _CB,),
    in_specs=[pl.BlockSpec((B, _CB, N), lambda i: (0, i, 0))],
    out_specs=pl.BlockSpec((_CB, N), lambda i: (i, 0)),
    out_shape=jax.ShapeDtypeStruct((C, N), jnp.float32),
)


_RB = 128


def _cov_body(xmb_ref, xm_ref, cov_ref, d2_ref):
    xmb = xmb_ref[...]
    xm = xm_ref[...]
    cov = lax.dot_general(xmb, xm, (((1,), (1,)), ((), ())),
                          preferred_element_type=jnp.float32) / (N - 1)
    cov_ref[...] = cov
    i0 = pl.program_id(0) * _RB
    ri = i0 + lax.broadcasted_iota(jnp.int32, (_RB, C), 0)
    ci = lax.broadcasted_iota(jnp.int32, (_RB, C), 1)
    d2_ref[...] = jnp.sum(jnp.where(ri == ci, cov, 0.0), axis=1,
                          keepdims=True)


_cov_call = pl.pallas_call(
    _cov_body,
    grid=(C // _RB,),
    in_specs=[pl.BlockSpec((_RB, N), lambda i: (i, 0)),
              pl.BlockSpec((C, N), lambda i: (0, 0))],
    out_specs=(pl.BlockSpec((_RB, C), lambda i: (i, 0)),
               pl.BlockSpec((_RB, 1), lambda i: (i, 0))),
    out_shape=(jax.ShapeDtypeStruct((C, C), jnp.float32),
               jax.ShapeDtypeStruct((C, 1), jnp.float32)),
)


def _ms_body(covb_ref, d2c_ref, d2r_ref, ms_ref):
    cov = covb_ref[...]
    dc = jnp.sqrt(d2c_ref[...])
    dr = jnp.sqrt(d2r_ref[...])
    corr = cov / (dc * dr)
    ms_ref[...] = jnp.mean(corr, axis=1, keepdims=True)


_ms_call = pl.pallas_call(
    _ms_body,
    grid=(C // _RB,),
    in_specs=[pl.BlockSpec((_RB, C), lambda i: (i, 0)),
              pl.BlockSpec((_RB, 1), lambda i: (i, 0)),
              pl.BlockSpec((1, C), lambda i: (0, 0))],
    out_specs=pl.BlockSpec((_RB, 1), lambda i: (i, 0)),
    out_shape=jax.ShapeDtypeStruct((C, 1), jnp.float32),
)


_PB = 32


def _pos_body(msr_ref, msc_ref, pos_ref):
    mj = msr_ref[...]
    mi = msc_ref[...]
    i0 = pl.program_id(0) * _PB
    ri = i0 + lax.broadcasted_iota(jnp.int32, (_PB, C), 0)
    ci = lax.broadcasted_iota(jnp.int32, (_PB, C), 1)
    posmat = (mj > mi) | ((mj == mi) & (ci < ri))
    pos_ref[...] = jnp.sum(posmat.astype(jnp.int32), axis=1, keepdims=True)


_pos_call = pl.pallas_call(
    _pos_body,
    grid=(C // _PB,),
    in_specs=[pl.BlockSpec((1, C), lambda i: (0, 0)),
              pl.BlockSpec((_PB, 1), lambda i: (i, 0))],
    out_specs=pl.BlockSpec((_PB, 1), lambda i: (i, 0)),
    out_shape=jax.ShapeDtypeStruct((C, 1), jnp.int32),
)


def _inv_body(posr_ref, sidx_ref):
    pr = posr_ref[...]
    p0 = pl.program_id(0) * _PB
    pi = p0 + lax.broadcasted_iota(jnp.int32, (_PB, C), 0)
    ci = lax.broadcasted_iota(jnp.int32, (_PB, C), 1)
    onehot = pr == pi
    sidx_ref[...] = jnp.sum(jnp.where(onehot, ci, 0), axis=1, keepdims=True)


_inv_call = pl.pallas_call(
    _inv_body,
    grid=(C // _PB,),
    in_specs=[pl.BlockSpec((1, C), lambda i: (0, 0))],
    out_specs=pl.BlockSpec((_PB, 1), lambda i: (i, 0)),
    out_shape=jax.ShapeDtypeStruct((C, 1), jnp.int32),
)


def _stats_call(query):
    xm = _xm_call(query)
    cov, d2 = _cov_call(xm, xm)
    ms = _ms_call(cov, d2, d2.reshape(1, C))
    pos = _pos_call(ms.reshape(1, C), ms)
    sidx = _inv_call(pos.reshape(1, C))
    return sidx.reshape(C)


def _gather_body(idx_hbm, q_hbm, k_hbm, v_hbm, *rest):
    outs = rest[:12]
    idx_v, buf, sem = rest[12:]
    w = lax.axis_index("s") * 2 + lax.axis_index("c")
    for t, src in enumerate((q_hbm, k_hbm, v_hbm)):
        for g in range(4):
            cnt = CNT[g]
            base = FLATOFF[g] + w * cnt
            iv = idx_v.at[pl.ds(0, cnt)]
            bv = buf.at[pl.ds(0, cnt)]
            pltpu.sync_copy(idx_hbm.at[pl.ds(base, cnt)], iv)
            pltpu.async_copy(src.at[iv], bv, sem).wait()
            pltpu.sync_copy(bv, outs[t * 4 + g].at[pl.ds(w * cnt, cnt)])


_gather_call_cache = []


def _gather_call(*args):
    if not _gather_call_cache:
        _gather_call_cache.append(pl.kernel(
            _gather_body,
            out_type=tuple(jax.ShapeDtypeStruct((8 * gs, N), jnp.float32)
                           for _ in range(3) for gs in GROUP_SIZES),
            mesh=plsc.VectorSubcoreMesh(core_axis_name="c",
                                        subcore_axis_name="s"),
            scratch_types=[
                pltpu.VMEM((96,), jnp.int32),
                pltpu.VMEM((96, N), jnp.float32),
                pltpu.SemaphoreType.DMA,
            ],
        ))
    return _gather_call_cache[0](*args)


def kernel(query, key, value):
    sidx = _stats_call(query)
    b_off = jnp.arange(B, dtype=jnp.int32) * C
    parts = []
    off = 0
    for gs in GROUP_SIZES:
        parts.append((sidx[off:off + gs][None, :] + b_off[:, None]).reshape(-1))
        off += gs
    idx_flat = jnp.concatenate(parts)
    q2 = query.reshape(B * C, N)
    k2 = key.reshape(B * C, N)
    v2 = value.reshape(B * C, N)
    outs = _gather_call(idx_flat, q2, k2, v2)
    res = []
    for t in range(3):
        res.append(tuple(outs[t * 4 + g].reshape(B, GROUP_SIZES[g], N)
                         for g in range(4)))
    return tuple(res)


# speedup vs baseline: 1.8248x; 1.0584x over previous
import jax
import jax.numpy as jnp
from jax import lax
from jax.experimental import pallas as pl
from jax.experimental.pallas import tpu as pltpu
from jax.experimental.pallas import tpu_sc as plsc

B, C, N = 8, 768, 1024
GROUP_SIZES = (96, 96, 192, 384)
FLATOFF = (0, 768, 1536, 3072)
NW = 32
CNT = tuple(8 * gs // NW for gs in GROUP_SIZES)


def _xm_body(q_ref, xm_ref):
    q = q_ref[...]
    cf = jnp.mean(q, axis=0)
    rm = jnp.mean(cf, axis=1, keepdims=True)
    xm_ref[...] = cf - rm


_CB = 128
_xm_call = pl.pallas_call(
    _xm_body,
    grid=(C // _CB,),
    in_specs=[pl.BlockSpec((B, _CB, N), lambda i: (0, i, 0))],
    out_specs=pl.BlockSpec((_CB, N), lambda i: (i, 0)),
    out_shape=jax.ShapeDtypeStruct((C, N), jnp.float32),
)


_RB = 128


def _cov_body(xmb_ref, xm_ref, cov_ref, d2_ref):
    xmb = xmb_ref[...]
    xm = xm_ref[...]
    cov = lax.dot_general(xmb, xm, (((1,), (1,)), ((), ())),
                          preferred_element_type=jnp.float32) / (N - 1)
    cov_ref[...] = cov
    i0 = pl.program_id(0) * _RB
    ri = i0 + lax.broadcasted_iota(jnp.int32, (_RB, C), 0)
    ci = lax.broadcasted_iota(jnp.int32, (_RB, C), 1)
    d2_ref[...] = jnp.sum(jnp.where(ri == ci, cov, 0.0), axis=1,
                          keepdims=True)


_cov_call = pl.pallas_call(
    _cov_body,
    grid=(C // _RB,),
    in_specs=[pl.BlockSpec((_RB, N), lambda i: (i, 0)),
              pl.BlockSpec((C, N), lambda i: (0, 0))],
    out_specs=(pl.BlockSpec((_RB, C), lambda i: (i, 0)),
               pl.BlockSpec((_RB, 1), lambda i: (i, 0))),
    out_shape=(jax.ShapeDtypeStruct((C, C), jnp.float32),
               jax.ShapeDtypeStruct((C, 1), jnp.float32)),
)


def _ms_body(covb_ref, d2c_ref, d2r_ref, ms_ref):
    cov = covb_ref[...]
    dc = jnp.sqrt(d2c_ref[...])
    dr = jnp.sqrt(d2r_ref[...])
    corr = cov / (dc * dr)
    ms_ref[...] = jnp.mean(corr, axis=1, keepdims=True)


_ms_call = pl.pallas_call(
    _ms_body,
    grid=(C // _RB,),
    in_specs=[pl.BlockSpec((_RB, C), lambda i: (i, 0)),
              pl.BlockSpec((_RB, 1), lambda i: (i, 0)),
              pl.BlockSpec((1, C), lambda i: (0, 0))],
    out_specs=pl.BlockSpec((_RB, 1), lambda i: (i, 0)),
    out_shape=jax.ShapeDtypeStruct((C, 1), jnp.float32),
)


_PB = 32


def _pos_body(msr_ref, msc_ref, pos_ref):
    mj = msr_ref[...]
    mi = msc_ref[...]
    i0 = pl.program_id(0) * _PB
    ri = i0 + lax.broadcasted_iota(jnp.int32, (_PB, C), 0)
    ci = lax.broadcasted_iota(jnp.int32, (_PB, C), 1)
    posmat = (mj > mi) | ((mj == mi) & (ci < ri))
    pos_ref[...] = jnp.sum(posmat.astype(jnp.int32), axis=1, keepdims=True)


_pos_call = pl.pallas_call(
    _pos_body,
    grid=(C // _PB,),
    in_specs=[pl.BlockSpec((1, C), lambda i: (0, 0)),
              pl.BlockSpec((_PB, 1), lambda i: (i, 0))],
    out_specs=pl.BlockSpec((_PB, 1), lambda i: (i, 0)),
    out_shape=jax.ShapeDtypeStruct((C, 1), jnp.int32),
)


def _inv_body(posr_ref, sidx_ref):
    pr = posr_ref[...]
    p0 = pl.program_id(0) * _PB
    pi = p0 + lax.broadcasted_iota(jnp.int32, (_PB, C), 0)
    ci = lax.broadcasted_iota(jnp.int32, (_PB, C), 1)
    onehot = pr == pi
    sidx_ref[...] = jnp.sum(jnp.where(onehot, ci, 0), axis=1, keepdims=True)


_inv_call = pl.pallas_call(
    _inv_body,
    grid=(C // _PB,),
    in_specs=[pl.BlockSpec((1, C), lambda i: (0, 0))],
    out_specs=pl.BlockSpec((_PB, 1), lambda i: (i, 0)),
    out_shape=jax.ShapeDtypeStruct((C, 1), jnp.int32),
)


def _stats_call(query):
    xm = _xm_call(query)
    cov, d2 = _cov_call(xm, xm)
    ms = _ms_call(cov, d2, d2.reshape(1, C))
    pos = _pos_call(ms.reshape(1, C), ms)
    sidx = _inv_call(pos.reshape(1, C))
    return sidx.reshape(C)


_CHUNKS = ((0, 0, 24), (1, 0, 24), (2, 0, 48), (3, 0, 48), (3, 48, 48))
_IDX_OFF = (0, 24, 48, 96)


def _gather_body(idx_hbm, q_hbm, k_hbm, v_hbm, *rest):
    outs = rest[:12]
    idx_v = rest[12]
    bufs = rest[13:15]
    gsems = rest[15:17]
    ssems = rest[17:19]
    w = lax.axis_index("s") * 2 + lax.axis_index("c")
    for g in range(4):
        cnt = CNT[g]
        pltpu.sync_copy(idx_hbm.at[pl.ds(FLATOFF[g] + w * cnt, cnt)],
                        idx_v.at[pl.ds(_IDX_OFF[g], cnt)])
    srcs = (q_hbm, k_hbm, v_hbm)
    jobs = [(t,) + ch for t in range(3) for ch in _CHUNKS]
    n = len(jobs)
    copies = [None] * n
    scat = [None] * n

    def _start_scatter(c):
        t, g, off, cn = jobs[c]
        scat[c] = pltpu.async_copy(
            bufs[c % 2].at[pl.ds(0, cn)],
            outs[t * 4 + g].at[pl.ds(w * CNT[g] + off, cn)],
            ssems[c % 2])

    for c in range(n):
        t, g, off, cn = jobs[c]
        b = c % 2
        if c >= 2:
            scat[c - 2].wait()
        copies[c] = pltpu.async_copy(
            srcs[t].at[idx_v.at[pl.ds(_IDX_OFF[g] + off, cn)]],
            bufs[b].at[pl.ds(0, cn)],
            gsems[b])
        if c >= 1:
            copies[c - 1].wait()
            _start_scatter(c - 1)
    copies[n - 1].wait()
    _start_scatter(n - 1)
    scat[n - 2].wait()
    scat[n - 1].wait()


_gather_call_cache = []


def _gather_call(*args):
    if not _gather_call_cache:
        _gather_call_cache.append(pl.kernel(
            _gather_body,
            out_type=tuple(jax.ShapeDtypeStruct((8 * gs, N), jnp.float32)
                           for _ in range(3) for gs in GROUP_SIZES),
            mesh=plsc.VectorSubcoreMesh(core_axis_name="c",
                                        subcore_axis_name="s"),
            scratch_types=[
                pltpu.VMEM((192,), jnp.int32),
                pltpu.VMEM((48, N), jnp.float32),
                pltpu.VMEM((48, N), jnp.float32),
                pltpu.SemaphoreType.DMA,
                pltpu.SemaphoreType.DMA,
                pltpu.SemaphoreType.DMA,
                pltpu.SemaphoreType.DMA,
            ],
        ))
    return _gather_call_cache[0](*args)


def kernel(query, key, value):
    sidx = _stats_call(query)
    b_off = jnp.arange(B, dtype=jnp.int32) * C
    parts = []
    off = 0
    for gs in GROUP_SIZES:
        parts.append((sidx[off:off + gs][None, :] + b_off[:, None]).reshape(-1))
        off += gs
    idx_flat = jnp.concatenate(parts)
    q2 = query.reshape(B * C, N)
    k2 = key.reshape(B * C, N)
    v2 = value.reshape(B * C, N)
    outs = _gather_call(idx_flat, q2, k2, v2)
    res = []
    for t in range(3):
        res.append(tuple(outs[t * 4 + g].reshape(B, GROUP_SIZES[g], N)
                         for g in range(4)))
    return tuple(res)


# speedup vs baseline: 1.8439x; 1.0105x over previous
import jax
import jax.numpy as jnp
from jax import lax
from jax.experimental import pallas as pl
from jax.experimental.pallas import tpu as pltpu
from jax.experimental.pallas import tpu_sc as plsc

B, C, N = 8, 768, 1024
GROUP_SIZES = (96, 96, 192, 384)
FLATOFF = (0, 768, 1536, 3072)
NW = 32
CNT = tuple(8 * gs // NW for gs in GROUP_SIZES)


def _xm_body(q_ref, xm_ref):
    q = q_ref[...]
    cf = jnp.mean(q, axis=0)
    rm = jnp.mean(cf, axis=1, keepdims=True)
    xm_ref[...] = cf - rm


_CB = 128
_xm_call = pl.pallas_call(
    _xm_body,
    grid=(C // _CB,),
    in_specs=[pl.BlockSpec((B, _CB, N), lambda i: (0, i, 0))],
    out_specs=pl.BlockSpec((_CB, N), lambda i: (i, 0)),
    out_shape=jax.ShapeDtypeStruct((C, N), jnp.float32),
)


_RB = 128


def _cov_body(xmb_ref, xm_ref, cov_ref, d2_ref):
    xmb = xmb_ref[...]
    xm = xm_ref[...]
    cov = lax.dot_general(xmb, xm, (((1,), (1,)), ((), ())),
                          preferred_element_type=jnp.float32) / (N - 1)
    cov_ref[...] = cov
    i0 = pl.program_id(0) * _RB
    ri = i0 + lax.broadcasted_iota(jnp.int32, (_RB, C), 0)
    ci = lax.broadcasted_iota(jnp.int32, (_RB, C), 1)
    d2_ref[...] = jnp.sum(jnp.where(ri == ci, cov, 0.0), axis=1,
                          keepdims=True)


_cov_call = pl.pallas_call(
    _cov_body,
    grid=(C // _RB,),
    in_specs=[pl.BlockSpec((_RB, N), lambda i: (i, 0)),
              pl.BlockSpec((C, N), lambda i: (0, 0))],
    out_specs=(pl.BlockSpec((_RB, C), lambda i: (i, 0)),
               pl.BlockSpec((_RB, 1), lambda i: (i, 0))),
    out_shape=(jax.ShapeDtypeStruct((C, C), jnp.float32),
               jax.ShapeDtypeStruct((C, 1), jnp.float32)),
)


def _ms_body(covb_ref, d2c_ref, d2r_ref, ms_ref):
    cov = covb_ref[...]
    dc = jnp.sqrt(d2c_ref[...])
    dr = jnp.sqrt(d2r_ref[...])
    corr = cov / (dc * dr)
    ms_ref[...] = jnp.mean(corr, axis=1, keepdims=True)


_ms_call = pl.pallas_call(
    _ms_body,
    grid=(C // _RB,),
    in_specs=[pl.BlockSpec((_RB, C), lambda i: (i, 0)),
              pl.BlockSpec((_RB, 1), lambda i: (i, 0)),
              pl.BlockSpec((1, C), lambda i: (0, 0))],
    out_specs=pl.BlockSpec((_RB, 1), lambda i: (i, 0)),
    out_shape=jax.ShapeDtypeStruct((C, 1), jnp.float32),
)


_PB = 32


def _pos_body(msr_ref, msc_ref, pos_ref):
    mj = msr_ref[...]
    mi = msc_ref[...]
    i0 = pl.program_id(0) * _PB
    ri = i0 + lax.broadcasted_iota(jnp.int32, (_PB, C), 0)
    ci = lax.broadcasted_iota(jnp.int32, (_PB, C), 1)
    posmat = (mj > mi) | ((mj == mi) & (ci < ri))
    pos_ref[...] = jnp.sum(posmat.astype(jnp.int32), axis=1, keepdims=True)


_pos_call = pl.pallas_call(
    _pos_body,
    grid=(C // _PB,),
    in_specs=[pl.BlockSpec((1, C), lambda i: (0, 0)),
              pl.BlockSpec((_PB, 1), lambda i: (i, 0))],
    out_specs=pl.BlockSpec((_PB, 1), lambda i: (i, 0)),
    out_shape=jax.ShapeDtypeStruct((C, 1), jnp.int32),
)


def _inv_body(posr_ref, sidx_ref):
    pr = posr_ref[...]
    p0 = pl.program_id(0) * _PB
    pi = p0 + lax.broadcasted_iota(jnp.int32, (_PB, C), 0)
    ci = lax.broadcasted_iota(jnp.int32, (_PB, C), 1)
    onehot = pr == pi
    sidx_ref[...] = jnp.sum(jnp.where(onehot, ci, 0), axis=1, keepdims=True)


_inv_call = pl.pallas_call(
    _inv_body,
    grid=(C // _PB,),
    in_specs=[pl.BlockSpec((1, C), lambda i: (0, 0))],
    out_specs=pl.BlockSpec((_PB, 1), lambda i: (i, 0)),
    out_shape=jax.ShapeDtypeStruct((C, 1), jnp.int32),
)


def _stats_call(query):
    xm = _xm_call(query)
    cov, d2 = _cov_call(xm, xm)
    ms = _ms_call(cov, d2, d2.reshape(1, C))
    pos = _pos_call(ms.reshape(1, C), ms)
    sidx = _inv_call(pos.reshape(1, C))
    return sidx.reshape(C)


_CK = 24
_CHUNKS = tuple((g, off) for g in range(4) for off in range(0, CNT[g], _CK))
_IDX_OFF = (0, 24, 48, 96)
_NB = 4


def _gather_body(idx_hbm, q_hbm, k_hbm, v_hbm, *rest):
    outs = rest[:12]
    idx_v = rest[12]
    bufs = rest[13:13 + _NB]
    gsems = rest[13 + _NB:13 + 2 * _NB]
    ssems = rest[13 + 2 * _NB:13 + 3 * _NB]
    w = lax.axis_index("s") * 2 + lax.axis_index("c")
    for g in range(4):
        cnt = CNT[g]
        pltpu.sync_copy(idx_hbm.at[pl.ds(FLATOFF[g] + w * cnt, cnt)],
                        idx_v.at[pl.ds(_IDX_OFF[g], cnt)])
    srcs = (q_hbm, k_hbm, v_hbm)
    jobs = [(t,) + ch for t in range(3) for ch in _CHUNKS]
    n = len(jobs)
    copies = [None] * n
    scat = [None] * n

    def _start_scatter(c):
        t, g, off = jobs[c]
        scat[c] = pltpu.async_copy(
            bufs[c % _NB],
            outs[t * 4 + g].at[pl.ds(w * CNT[g] + off, _CK)],
            ssems[c % _NB])

    for c in range(n):
        t, g, off = jobs[c]
        b = c % _NB
        if c >= _NB:
            scat[c - _NB].wait()
        copies[c] = pltpu.async_copy(
            srcs[t].at[idx_v.at[pl.ds(_IDX_OFF[g] + off, _CK)]],
            bufs[b],
            gsems[b])
        if c >= 1:
            copies[c - 1].wait()
            _start_scatter(c - 1)
    copies[n - 1].wait()
    _start_scatter(n - 1)
    for c in range(n - _NB, n):
        scat[c].wait()


_gather_call_cache = []


def _gather_call(*args):
    if not _gather_call_cache:
        _gather_call_cache.append(pl.kernel(
            _gather_body,
            out_type=tuple(jax.ShapeDtypeStruct((8 * gs, N), jnp.float32)
                           for _ in range(3) for gs in GROUP_SIZES),
            mesh=plsc.VectorSubcoreMesh(core_axis_name="c",
                                        subcore_axis_name="s"),
            scratch_types=(
                [pltpu.VMEM((192,), jnp.int32)]
                + [pltpu.VMEM((_CK, N), jnp.float32) for _ in range(_NB)]
                + [pltpu.SemaphoreType.DMA for _ in range(2 * _NB)]
            ),
        ))
    return _gather_call_cache[0](*args)


def kernel(query, key, value):
    sidx = _stats_call(query)
    b_off = jnp.arange(B, dtype=jnp.int32) * C
    parts = []
    off = 0
    for gs in GROUP_SIZES:
        parts.append((sidx[off:off + gs][None, :] + b_off[:, None]).reshape(-1))
        off += gs
    idx_flat = jnp.concatenate(parts)
    q2 = query.reshape(B * C, N)
    k2 = key.reshape(B * C, N)
    v2 = value.reshape(B * C, N)
    outs = _gather_call(idx_flat, q2, k2, v2)
    res = []
    for t in range(3):
        res.append(tuple(outs[t * 4 + g].reshape(B, GROUP_SIZES[g], N)
                         for g in range(4)))
    return tuple(res)


# speedup vs baseline: 2.3922x; 1.2973x over previous
import jax
import jax.numpy as jnp
from jax import lax
from jax.experimental import pallas as pl
from jax.experimental.pallas import tpu as pltpu
from jax.experimental.pallas import tpu_sc as plsc

B, C, N = 8, 768, 1024
GROUP_SIZES = (96, 96, 192, 384)
FLATOFF = (0, 768, 1536, 3072)
NW = 32
CNT = tuple(8 * gs // NW for gs in GROUP_SIZES)


_CB = 128
_PB = 32


def _k1_body(q_ref, ms_ref, xm_s, cov_s, d2c_s, d2r_s):
    i = pl.program_id(0)
    ph = i // 6
    blk = i % 6
    i0 = blk * _CB

    @pl.when(ph == 0)
    def _xm():
        q = q_ref[...]
        cf = jnp.mean(q, axis=0)
        rm = jnp.mean(cf, axis=1, keepdims=True)
        xm_s[pl.ds(i0, _CB), :] = cf - rm

    @pl.when(ph == 1)
    def _cov():
        xmb = xm_s[pl.ds(i0, _CB), :]
        xm = xm_s[...]
        cov = lax.dot_general(xmb, xm, (((1,), (1,)), ((), ())),
                              preferred_element_type=jnp.float32) / (N - 1)
        cov_s[pl.ds(i0, _CB), :] = cov
        ri = i0 + lax.broadcasted_iota(jnp.int32, (_CB, C), 0)
        ci = lax.broadcasted_iota(jnp.int32, (_CB, C), 1)
        diag = jnp.where(ri == ci, cov, 0.0)
        d2c_s[pl.ds(i0, _CB), :] = jnp.sum(diag, axis=1, keepdims=True)
        part_r = jnp.sum(diag, axis=0, keepdims=True)
        prev = jnp.where(blk == 0, jnp.zeros_like(part_r), d2r_s[...])
        d2r_s[...] = prev + part_r

    @pl.when(ph == 2)
    def _ms():
        cov = cov_s[pl.ds(i0, _CB), :]
        dc = jnp.sqrt(d2c_s[pl.ds(i0, _CB), :])
        dr = jnp.sqrt(d2r_s[...])
        corr = cov / (dc * dr)
        ms_ref[...] = jnp.mean(corr, axis=1, keepdims=True)


_k1_call = pl.pallas_call(
    _k1_body,
    grid=(18,),
    in_specs=[pl.BlockSpec((B, _CB, N), lambda i: (0, jnp.minimum(i, 5), 0))],
    out_specs=pl.BlockSpec((_CB, 1),
                           lambda i: (jnp.where(i >= 12, i - 12, 0), 0)),
    out_shape=jax.ShapeDtypeStruct((C, 1), jnp.float32),
    scratch_shapes=[
        pltpu.VMEM((C, N), jnp.float32),
        pltpu.VMEM((C, C), jnp.float32),
        pltpu.VMEM((C, 1), jnp.float32),
        pltpu.VMEM((1, C), jnp.float32),
    ],
)


def _k2_body(msr_ref, msc_ref, idx2_ref, sidx_s):
    i = pl.program_id(0)

    @pl.when(i < 24)
    def _posinv():
        i0 = i * _PB
        mj = msr_ref[...]
        mi = msc_ref[pl.ds(i0, _PB), :]
        ri = i0 + lax.broadcasted_iota(jnp.int32, (_PB, C), 0)
        ci = lax.broadcasted_iota(jnp.int32, (_PB, C), 1)
        posmat = (mj > mi) | ((mj == mi) & (ci < ri))
        pos = jnp.sum(posmat.astype(jnp.int32), axis=1, keepdims=True)
        part = jnp.sum(jnp.where(pos == ci, ri, 0), axis=0, keepdims=True)
        prev = jnp.where(i == 0, jnp.zeros_like(part), sidx_s[...])
        sidx_s[...] = prev + part

    @pl.when(i == 24)
    def _emit():
        bi = lax.broadcasted_iota(jnp.int32, (B, C), 0) * C
        idx2_ref[...] = bi + sidx_s[...]


_k2_call = pl.pallas_call(
    _k2_body,
    grid=(25,),
    in_specs=[pl.BlockSpec((1, C), lambda i: (0, 0)),
              pl.BlockSpec((C, 1), lambda i: (0, 0))],
    out_specs=pl.BlockSpec((B, C), lambda i: (0, 0)),
    out_shape=jax.ShapeDtypeStruct((B, C), jnp.int32),
    scratch_shapes=[pltpu.VMEM((1, C), jnp.int32)],
)


def _stats_call(query):
    ms = _k1_call(query)
    idx2 = _k2_call(ms.reshape(1, C), ms)
    return idx2.reshape(B * C)


_CK = 24
_CHUNKS = tuple((g, off) for g in range(4) for off in range(0, CNT[g], _CK))
_IDX_OFF = (0, 24, 48, 96)
_OFFG = (0, 96, 192, 384)
_NB = 4


def _gather_body(idx_hbm, q_hbm, k_hbm, v_hbm, *rest):
    outs = rest[:12]
    idx_v = rest[12]
    bufs = rest[13:13 + _NB]
    gsems = rest[13 + _NB:13 + 2 * _NB]
    ssems = rest[13 + 2 * _NB:13 + 3 * _NB]
    w = lax.axis_index("s") * 2 + lax.axis_index("c")
    bb = w // 4
    for g in range(4):
        cnt = CNT[g]
        base = bb * C + _OFFG[g] + (w * cnt - bb * GROUP_SIZES[g])
        pltpu.sync_copy(idx_hbm.at[pl.ds(base, cnt)],
                        idx_v.at[pl.ds(_IDX_OFF[g], cnt)])
    srcs = (q_hbm, k_hbm, v_hbm)
    jobs = [(t,) + ch for t in range(3) for ch in _CHUNKS]
    n = len(jobs)
    copies = [None] * n
    scat = [None] * n

    def _start_scatter(c):
        t, g, off = jobs[c]
        scat[c] = pltpu.async_copy(
            bufs[c % _NB],
            outs[t * 4 + g].at[pl.ds(w * CNT[g] + off, _CK)],
            ssems[c % _NB])

    for c in range(n):
        t, g, off = jobs[c]
        b = c % _NB
        if c >= _NB:
            scat[c - _NB].wait()
        copies[c] = pltpu.async_copy(
            srcs[t].at[idx_v.at[pl.ds(_IDX_OFF[g] + off, _CK)]],
            bufs[b],
            gsems[b])
        if c >= 1:
            copies[c - 1].wait()
            _start_scatter(c - 1)
    copies[n - 1].wait()
    _start_scatter(n - 1)
    for c in range(n - _NB, n):
        scat[c].wait()


_gather_call_cache = []


def _gather_call(*args):
    if not _gather_call_cache:
        _gather_call_cache.append(pl.kernel(
            _gather_body,
            out_type=tuple(jax.ShapeDtypeStruct((8 * gs, N), jnp.float32)
                           for _ in range(3) for gs in GROUP_SIZES),
            mesh=plsc.VectorSubcoreMesh(core_axis_name="c",
                                        subcore_axis_name="s"),
            scratch_types=(
                [pltpu.VMEM((192,), jnp.int32)]
                + [pltpu.VMEM((_CK, N), jnp.float32) for _ in range(_NB)]
                + [pltpu.SemaphoreType.DMA for _ in range(2 * _NB)]
            ),
        ))
    return _gather_call_cache[0](*args)


def kernel(query, key, value):
    idx_flat = _stats_call(query)
    q2 = query.reshape(B * C, N)
    k2 = key.reshape(B * C, N)
    v2 = value.reshape(B * C, N)
    outs = _gather_call(idx_flat, q2, k2, v2)
    res = []
    for t in range(3):
        res.append(tuple(outs[t * 4 + g].reshape(B, GROUP_SIZES[g], N)
                         for g in range(4)))
    return tuple(res)
